# Initial kernel scaffold; baseline (speedup 1.0000x reference)
import jax
import jax.numpy as jnp
from jax.experimental import pallas as pl


def kernel(boxes, scores, idxs):
    raise NotImplementedError("write your pallas kernel here")


# speedup vs baseline: 54.9851x; 54.9851x over previous
import functools

import jax
import jax.numpy as jnp
from jax import lax
from jax.experimental import pallas as pl
from jax.experimental.pallas import tpu as pltpu
from jax.experimental.pallas import tpu_sc as plsc

N = 20000
NPAD = 20480
THR = 0.5

RB = 128
CB = 4096
WPC = CB // 32
NW = NPAD // 32
NRB = NPAD // RB
NCB = NPAD // CB

GB = 32


def _pairs_body(x1r, y1r, x2r, y2r, ar, x1c, y1c, x2c, y2c, ac, m_ref, nz_ref):
    r = pl.program_id(0)
    c = pl.program_id(1)
    has_upper = c * CB + (CB - 1) > r * RB

    @pl.when(has_upper)
    def _compute():
        xx1 = jnp.maximum(x1r[...], x1c[...])
        yy1 = jnp.maximum(y1r[...], y1c[...])
        xx2 = jnp.minimum(x2r[...], x2c[...])
        yy2 = jnp.minimum(y2r[...], y2c[...])
        w = jnp.maximum(xx2 - xx1 + 1.0, 0.0)
        h = jnp.maximum(yy2 - yy1 + 1.0, 0.0)
        inter = w * h
        o = inter / (ar[...] + ac[...] - inter)
        ivec = lax.broadcasted_iota(jnp.int32, (RB, CB), 0) + r * RB
        jvec = lax.broadcasted_iota(jnp.int32, (RB, CB), 1) + c * CB
        sup = (o > THR) & (jvec > ivec)
        acc = jnp.zeros((RB, WPC), jnp.uint32)
        for b in range(32):
            acc = acc | jnp.where(
                sup[:, b * WPC:(b + 1) * WPC],
                jnp.uint32(1 << b), jnp.uint32(0))
        m_ref[...] = acc
        nz_ref[...] = jnp.any(acc != 0, axis=1, keepdims=True).astype(
            jnp.int32).reshape(1, RB, 1)

    @pl.when(jnp.logical_not(has_upper))
    def _zero():
        m_ref[...] = jnp.zeros((RB, WPC), jnp.uint32)
        nz_ref[...] = jnp.zeros((1, RB, 1), jnp.int32)


def _run_pairs(x1, y1, x2, y2, area, interpret=False):
    row = lambda a: a.reshape(NPAD, 1)
    col = lambda a: a.reshape(1, NPAD)
    rspec = pl.BlockSpec((RB, 1), lambda r, c: (r, 0))
    cspec = pl.BlockSpec((1, CB), lambda r, c: (0, c))
    return pl.pallas_call(
        _pairs_body,
        grid=(NRB, NCB),
        in_specs=[rspec] * 5 + [cspec] * 5,
        out_specs=[
            pl.BlockSpec((RB, WPC), lambda r, c: (r, c)),
            pl.BlockSpec((1, RB, 1), lambda r, c: (c, r, 0)),
        ],
        out_shape=[
            jax.ShapeDtypeStruct((NPAD, NW), jnp.uint32),
            jax.ShapeDtypeStruct((NCB, NPAD, 1), jnp.int32),
        ],
        interpret=interpret,
    )(row(x1), row(y1), row(x2), row(y2), row(area),
      col(x1), col(y1), col(x2), col(y2), col(area))


def _scan_body(m_hbm, rnz_hbm, out_hbm, idxbuf, rnzbuf, idxbatch, rowbuf,
               removed, sem):
    cid = lax.axis_index("c")
    sid = lax.axis_index("s")

    @pl.when(jnp.logical_and(cid == 0, sid == 0))
    def _():
        pltpu.sync_copy(rnz_hbm, rnzbuf)

        zero16 = jnp.zeros((16,), jnp.uint32)

        def zb(w, carry):
            removed[pl.ds(w * 16, 16)] = zero16
            return carry

        lax.fori_loop(0, (NW + 16) // 16, zb, 0)

        def cb(v, cnt):
            vec = rnzbuf[pl.ds(v * 16, 16)]
            mask = vec != 0
            ids = lax.iota(jnp.int32, 16) + v * 16
            mi = mask.astype(jnp.int32)
            pos = cnt + plsc.cumsum(mi) - 1
            plsc.store_scatter(idxbuf, [pos], ids, mask=mask)
            return cnt + jnp.sum(mi)

        cnt = lax.fori_loop(0, NPAD // 16, cb, jnp.int32(0))

        idxbuf[pl.ds(cnt, 16)] = jnp.zeros((16,), jnp.int32)
        idxbuf[pl.ds(cnt + 16, 16)] = jnp.zeros((16,), jnp.int32)

        nb = (cnt + GB - 1) // GB

        def batch(t, carry):
            for h in range(GB // 16):
                idxbatch[pl.ds(h * 16, 16)] = idxbuf[pl.ds(t * GB + h * 16, 16)]
            pltpu.async_copy(m_hbm.at[idxbatch], rowbuf, sem)
            pltpu.make_async_copy(m_hbm.at[idxbatch], rowbuf, sem).wait()
            kmax = jnp.minimum(GB, cnt - t * GB)

            def sb(k, c2):
                i = idxbuf[pl.ds(t * GB + k, 16)][0]
                word = (i >> 12) * WPC + (i & (WPC - 1))
                bit = ((i & (CB - 1)) >> 7).astype(jnp.uint32)
                wv = removed[pl.ds(word, 16)][0]
                alive = ((wv >> bit) & jnp.uint32(1)) == 0

                @pl.when(alive)
                def _or():
                    ws = (i >> 12) * (WPC // 16)

                    def ob(w, c3):
                        sl = pl.ds(w * 16, 16)
                        removed[sl] = removed[sl] | rowbuf[k, sl]
                        return c3

                    lax.fori_loop(ws, NW // 16, ob, 0)

                return c2

            lax.fori_loop(0, kmax, sb, 0)
            return carry

        lax.fori_loop(0, nb, batch, 0)
        pltpu.sync_copy(removed.at[pl.ds(0, NW)], out_hbm)


def _run_scan(m, rownz):
    mesh = plsc.VectorSubcoreMesh(core_axis_name="c", subcore_axis_name="s")
    fn = pl.kernel(
        _scan_body,
        out_type=jax.ShapeDtypeStruct((NW,), jnp.uint32),
        mesh=mesh,
        compiler_params=pltpu.CompilerParams(needs_layout_passes=False),
        scratch_types=[
            pltpu.VMEM((NPAD + 32,), jnp.int32),
            pltpu.VMEM((NPAD,), jnp.int32),
            pltpu.VMEM((GB,), jnp.int32),
            pltpu.VMEM((GB, NW), jnp.uint32),
            pltpu.VMEM((NW + 16,), jnp.uint32),
            pltpu.SemaphoreType.DMA,
        ],
    )
    return fn(m, rownz)


def kernel(boxes, scores, idxs):
    max_coordinate = jnp.max(boxes)
    offsets = idxs.astype(boxes.dtype) * (max_coordinate + 1.0)
    boxes_for_nms = boxes + offsets[:, None]
    order = jnp.argsort(-scores)
    bs = boxes_for_nms[order]
    padbox = jnp.array([0.0, 0.0, -10.0, -10.0], jnp.float32)
    bp = jnp.concatenate(
        [bs, jnp.broadcast_to(padbox, (NPAD - N, 4))], axis=0)
    x1, y1, x2, y2 = bp[:, 0], bp[:, 1], bp[:, 2], bp[:, 3]
    area = (x2 - x1 + 1.0) * (y2 - y1 + 1.0)

    m, nz = _run_pairs(x1, y1, x2, y2, area)
    rownz = jnp.any(nz[:, :, 0] != 0, axis=0).astype(jnp.int32)
    removed = _run_scan(m, rownz)

    i = jnp.arange(N, dtype=jnp.int32)
    word = (i >> 12) * WPC + (i & (WPC - 1))
    bit = ((i & (CB - 1)) >> 7).astype(jnp.uint32)
    keep_sorted = ((removed[word] >> bit) & jnp.uint32(1)) == 0
    kept_boxes = jnp.where(keep_sorted[:, None], boxes[order], 0.0)
    kept_scores = jnp.where(keep_sorted, scores[order], 0.0)
    return kept_boxes, kept_scores, keep_sorted


# speedup vs baseline: 57.2500x; 1.0412x over previous
import functools

import jax
import jax.numpy as jnp
from jax import lax
from jax.experimental import pallas as pl
from jax.experimental.pallas import tpu as pltpu
from jax.experimental.pallas import tpu_sc as plsc

N = 20000
NPAD = 20480
THR = 0.5

RB = 128
CB = 4096
WPC = CB // 32
NW = NPAD // 32
NRB = NPAD // RB
NCB = NPAD // CB

GB = 64


def _pairs_body(x1r, y1r, x2r, y2r, ar, x1c, y1c, x2c, y2c, ac, m_ref, nz_ref):
    r = pl.program_id(0)
    c = pl.program_id(1)
    full_upper = c * CB > r * RB + (RB - 1)
    full_lower = c * CB + (CB - 1) <= r * RB

    def emit(masked):
        xx1 = jnp.maximum(x1r[...], x1c[...])
        yy1 = jnp.maximum(y1r[...], y1c[...])
        xx2 = jnp.minimum(x2r[...], x2c[...])
        yy2 = jnp.minimum(y2r[...], y2c[...])
        w = jnp.maximum(xx2 - xx1 + 1.0, 0.0)
        h = jnp.maximum(yy2 - yy1 + 1.0, 0.0)
        inter = w * h
        o = inter / (ar[...] + ac[...] - inter)
        sup = o > THR
        if masked:
            ivec = lax.broadcasted_iota(jnp.int32, (RB, CB), 0) + r * RB
            jvec = lax.broadcasted_iota(jnp.int32, (RB, CB), 1) + c * CB
            sup = sup & (jvec > ivec)
        acc = jnp.zeros((RB, WPC), jnp.uint32)
        for b in range(32):
            acc = acc | jnp.where(
                sup[:, b * WPC:(b + 1) * WPC],
                jnp.uint32(1 << b), jnp.uint32(0))
        m_ref[...] = acc
        nz_ref[...] = jnp.any(acc != 0, axis=1, keepdims=True).astype(
            jnp.int32).reshape(1, RB, 1)

    @pl.when(full_upper)
    def _upper():
        emit(masked=False)

    @pl.when(jnp.logical_not(full_upper | full_lower))
    def _diag():
        emit(masked=True)

    @pl.when(full_lower)
    def _zero():
        m_ref[...] = jnp.zeros((RB, WPC), jnp.uint32)
        nz_ref[...] = jnp.zeros((1, RB, 1), jnp.int32)


def _run_pairs(x1, y1, x2, y2, area, interpret=False):
    row = lambda a: a.reshape(NPAD, 1)
    col = lambda a: a.reshape(1, NPAD)
    rspec = pl.BlockSpec((RB, 1), lambda r, c: (r, 0))
    cspec = pl.BlockSpec((1, CB), lambda r, c: (0, c))
    return pl.pallas_call(
        _pairs_body,
        grid=(NRB, NCB),
        in_specs=[rspec] * 5 + [cspec] * 5,
        out_specs=[
            pl.BlockSpec((RB, WPC), lambda r, c: (r, c)),
            pl.BlockSpec((1, RB, 1), lambda r, c: (c, r, 0)),
        ],
        out_shape=[
            jax.ShapeDtypeStruct((NPAD, NW), jnp.uint32),
            jax.ShapeDtypeStruct((NCB, NPAD, 1), jnp.int32),
        ],
        interpret=interpret,
    )(row(x1), row(y1), row(x2), row(y2), row(area),
      col(x1), col(y1), col(x2), col(y2), col(area))


def _scan_body(m_hbm, rnz_hbm, out_hbm, idxbuf, rnzbuf, idxbatch, rowbuf,
               removed, sem):
    cid = lax.axis_index("c")
    sid = lax.axis_index("s")

    @pl.when(jnp.logical_and(cid == 0, sid == 0))
    def _():
        pltpu.sync_copy(rnz_hbm, rnzbuf)

        zero16 = jnp.zeros((16,), jnp.uint32)

        def zb(w, carry):
            removed[pl.ds(w * 16, 16)] = zero16
            return carry

        lax.fori_loop(0, (NW + 16) // 16, zb, 0)

        def cb(v, cnt):
            vec = rnzbuf[pl.ds(v * 16, 16)]
            mask = vec != 0
            ids = lax.iota(jnp.int32, 16) + v * 16
            mi = mask.astype(jnp.int32)
            pos = cnt + plsc.cumsum(mi) - 1
            plsc.store_scatter(idxbuf, [pos], ids, mask=mask)
            return cnt + jnp.sum(mi)

        cnt = lax.fori_loop(0, NPAD // 16, cb, jnp.int32(0))

        idxbuf[pl.ds(cnt, 16)] = jnp.zeros((16,), jnp.int32)
        idxbuf[pl.ds(cnt + 16, 16)] = jnp.zeros((16,), jnp.int32)

        nb = (cnt + GB - 1) // GB

        def stage(t, buf):
            for h in range(GB // 16):
                idxbatch[buf, pl.ds(h * 16, 16)] = (
                    idxbuf[pl.ds(t * GB + h * 16, 16)])
            pltpu.async_copy(
                m_hbm.at[idxbatch.at[buf]], rowbuf.at[buf], sem.at[buf])

        @pl.when(nb > 0)
        def _prime():
            stage(0, 0)

        def batch(t, carry):
            buf = lax.rem(t, 2)

            @pl.when(t + 1 < nb)
            def _ahead():
                stage(t + 1, 1 - buf)

            pltpu.make_async_copy(
                m_hbm.at[idxbatch.at[buf]], rowbuf.at[buf], sem.at[buf]).wait()
            kmax = jnp.minimum(GB, cnt - t * GB)

            def sb(k, c2):
                i = idxbuf[pl.ds(t * GB + k, 16)][0]
                word = (i >> 12) * WPC + (i & (WPC - 1))
                bit = ((i & (CB - 1)) >> 7).astype(jnp.uint32)
                wv = removed[pl.ds(word, 16)][0]
                alive = ((wv >> bit) & jnp.uint32(1)) == 0

                @pl.when(alive)
                def _or():
                    ws = (i >> 12) * (WPC // 16)

                    def ob(w, c3):
                        sl = pl.ds(w * 16, 16)
                        removed[sl] = removed[sl] | rowbuf[buf, k, sl]
                        return c3

                    lax.fori_loop(ws, NW // 16, ob, 0)

                return c2

            lax.fori_loop(0, kmax, sb, 0)
            return carry

        lax.fori_loop(0, nb, batch, 0)
        pltpu.sync_copy(removed.at[pl.ds(0, NW)], out_hbm)


def _run_scan(m, rownz):
    mesh = plsc.VectorSubcoreMesh(core_axis_name="c", subcore_axis_name="s")
    fn = pl.kernel(
        _scan_body,
        out_type=jax.ShapeDtypeStruct((NW,), jnp.uint32),
        mesh=mesh,
        compiler_params=pltpu.CompilerParams(needs_layout_passes=False),
        scratch_types=[
            pltpu.VMEM((NPAD + 32,), jnp.int32),
            pltpu.VMEM((NPAD,), jnp.int32),
            pltpu.VMEM((2, GB), jnp.int32),
            pltpu.VMEM((2, GB, NW), jnp.uint32),
            pltpu.VMEM((NW + 16,), jnp.uint32),
            pltpu.SemaphoreType.DMA((2,)),
        ],
    )
    return fn(m, rownz)


def kernel(boxes, scores, idxs):
    max_coordinate = jnp.max(boxes)
    offsets = idxs.astype(boxes.dtype) * (max_coordinate + 1.0)
    boxes_for_nms = boxes + offsets[:, None]
    order = jnp.argsort(-scores)
    bs = boxes_for_nms[order]
    padbox = jnp.array([0.0, 0.0, -10.0, -10.0], jnp.float32)
    bp = jnp.concatenate(
        [bs, jnp.broadcast_to(padbox, (NPAD - N, 4))], axis=0)
    x1, y1, x2, y2 = bp[:, 0], bp[:, 1], bp[:, 2], bp[:, 3]
    area = (x2 - x1 + 1.0) * (y2 - y1 + 1.0)

    m, nz = _run_pairs(x1, y1, x2, y2, area)
    rownz = jnp.any(nz[:, :, 0] != 0, axis=0).astype(jnp.int32)
    removed = _run_scan(m, rownz)

    i = jnp.arange(N, dtype=jnp.int32)
    word = (i >> 12) * WPC + (i & (WPC - 1))
    bit = ((i & (CB - 1)) >> 7).astype(jnp.uint32)
    keep_sorted = ((removed[word] >> bit) & jnp.uint32(1)) == 0
    kept_boxes = jnp.where(keep_sorted[:, None], boxes[order], 0.0)
    kept_scores = jnp.where(keep_sorted, scores[order], 0.0)
    return kept_boxes, kept_scores, keep_sorted


# speedup vs baseline: 62.5830x; 1.0932x over previous
import functools

import jax
import jax.numpy as jnp
from jax import lax
from jax.experimental import pallas as pl
from jax.experimental.pallas import tpu as pltpu
from jax.experimental.pallas import tpu_sc as plsc

N = 20000
NPAD = 20480
THR = 0.5

RB = 256
CB = 4096
WPC = CB // 32
NW = NPAD // 32
NRB = NPAD // RB
NCB = NPAD // CB

GB = 64


def _pairs_body(x1r, y1r, x2r, y2r, ar, x1c, y1c, x2c, y2c, ac, m_ref, nz_ref):
    r = pl.program_id(0)
    c = pl.program_id(1)
    full_upper = c * CB > r * RB + (RB - 1)
    full_lower = c * CB + (CB - 1) <= r * RB

    def emit(masked):
        xx1 = jnp.maximum(x1r[...], x1c[...])
        yy1 = jnp.maximum(y1r[...], y1c[...])
        xx2 = jnp.minimum(x2r[...], x2c[...])
        yy2 = jnp.minimum(y2r[...], y2c[...])
        w = jnp.maximum(xx2 - xx1 + 1.0, 0.0)
        h = jnp.maximum(yy2 - yy1 + 1.0, 0.0)
        inter = w * h
        o = inter / (ar[...] + ac[...] - inter)
        sup = o > THR
        if masked:
            ivec = lax.broadcasted_iota(jnp.int32, (RB, CB), 0) + r * RB
            jvec = lax.broadcasted_iota(jnp.int32, (RB, CB), 1) + c * CB
            sup = sup & (jvec > ivec)
        acc = jnp.zeros((RB, WPC), jnp.uint32)
        for b in range(32):
            acc = acc | jnp.where(
                sup[:, b * WPC:(b + 1) * WPC],
                jnp.uint32(1 << b), jnp.uint32(0))
        m_ref[...] = acc
        nz_ref[...] = jnp.any(acc != 0, axis=1, keepdims=True).astype(
            jnp.int32).reshape(1, RB, 1)

    @pl.when(full_upper)
    def _upper():
        emit(masked=False)

    @pl.when(jnp.logical_not(full_upper | full_lower))
    def _diag():
        emit(masked=True)

    @pl.when(full_lower)
    def _zero():
        m_ref[...] = jnp.zeros((RB, WPC), jnp.uint32)
        nz_ref[...] = jnp.zeros((1, RB, 1), jnp.int32)


def _run_pairs(x1, y1, x2, y2, area, interpret=False):
    row = lambda a: a.reshape(NPAD, 1)
    col = lambda a: a.reshape(1, NPAD)
    rspec = pl.BlockSpec((RB, 1), lambda r, c: (r, 0))
    cspec = pl.BlockSpec((1, CB), lambda r, c: (0, c))
    return pl.pallas_call(
        _pairs_body,
        grid=(NRB, NCB),
        in_specs=[rspec] * 5 + [cspec] * 5,
        out_specs=[
            pl.BlockSpec((RB, WPC), lambda r, c: (r, c)),
            pl.BlockSpec((1, RB, 1), lambda r, c: (c, r, 0)),
        ],
        out_shape=[
            jax.ShapeDtypeStruct((NPAD, NW), jnp.uint32),
            jax.ShapeDtypeStruct((NCB, NPAD, 1), jnp.int32),
        ],
        interpret=interpret,
    )(row(x1), row(y1), row(x2), row(y2), row(area),
      col(x1), col(y1), col(x2), col(y2), col(area))


def _scan_body(m_hbm, rnz_hbm, out_hbm, idxbuf, rnzbuf, idxbatch, rowbuf,
               removed, sem):
    cid = lax.axis_index("c")
    sid = lax.axis_index("s")

    @pl.when(jnp.logical_and(cid == 0, sid == 0))
    def _():
        pltpu.sync_copy(rnz_hbm, rnzbuf)

        zero16 = jnp.zeros((16,), jnp.uint32)

        def zb(w, carry):
            removed[pl.ds(w * 16, 16)] = zero16
            return carry

        lax.fori_loop(0, (NW + 16) // 16, zb, 0)

        def cb(v, cnt):
            vec = rnzbuf[pl.ds(v * 16, 16)]
            mask = vec != 0
            ids = lax.iota(jnp.int32, 16) + v * 16
            mi = mask.astype(jnp.int32)
            pos = cnt + plsc.cumsum(mi) - 1
            plsc.store_scatter(idxbuf, [pos], ids, mask=mask)
            return cnt + jnp.sum(mi)

        cnt = lax.fori_loop(0, NPAD // 16, cb, jnp.int32(0))

        idxbuf[pl.ds(cnt, 16)] = jnp.zeros((16,), jnp.int32)
        idxbuf[pl.ds(cnt + 16, 16)] = jnp.zeros((16,), jnp.int32)

        nb = (cnt + GB - 1) // GB

        def stage(t, buf):
            for h in range(GB // 16):
                idxbatch[buf, pl.ds(h * 16, 16)] = (
                    idxbuf[pl.ds(t * GB + h * 16, 16)])
            pltpu.async_copy(
                m_hbm.at[idxbatch.at[buf]], rowbuf.at[buf], sem.at[buf])

        @pl.when(nb > 0)
        def _prime():
            stage(0, 0)

        def batch(t, carry):
            buf = lax.rem(t, 2)

            @pl.when(t + 1 < nb)
            def _ahead():
                stage(t + 1, 1 - buf)

            pltpu.make_async_copy(
                m_hbm.at[idxbatch.at[buf]], rowbuf.at[buf], sem.at[buf]).wait()
            kmax = jnp.minimum(GB, cnt - t * GB)

            def sb(k, c2):
                i = idxbuf[pl.ds(t * GB + k, 16)][0]
                word = (i >> 12) * WPC + (i & (WPC - 1))
                bit = ((i & (CB - 1)) >> 7).astype(jnp.uint32)
                wv = removed[pl.ds(word, 16)][0]
                alive = ((wv >> bit) & jnp.uint32(1)) == 0

                @pl.when(alive)
                def _or():
                    ws = (i >> 12) * (WPC // 16)

                    def ob(w, c3):
                        sl = pl.ds(w * 16, 16)
                        removed[sl] = removed[sl] | rowbuf[buf, k, sl]
                        return c3

                    lax.fori_loop(ws, NW // 16, ob, 0)

                return c2

            lax.fori_loop(0, kmax, sb, 0)
            return carry

        lax.fori_loop(0, nb, batch, 0)
        pltpu.sync_copy(removed.at[pl.ds(0, NW)], out_hbm)


def _run_scan(m, rownz):
    mesh = plsc.VectorSubcoreMesh(core_axis_name="c", subcore_axis_name="s")
    fn = pl.kernel(
        _scan_body,
        out_type=jax.ShapeDtypeStruct((NW,), jnp.uint32),
        mesh=mesh,
        compiler_params=pltpu.CompilerParams(needs_layout_passes=False),
        scratch_types=[
            pltpu.VMEM((NPAD + 32,), jnp.int32),
            pltpu.VMEM((NPAD,), jnp.int32),
            pltpu.VMEM((2, GB), jnp.int32),
            pltpu.VMEM((2, GB, NW), jnp.uint32),
            pltpu.VMEM((NW + 16,), jnp.uint32),
            pltpu.SemaphoreType.DMA((2,)),
        ],
    )
    return fn(m, rownz)


def kernel(boxes, scores, idxs):
    max_coordinate = jnp.max(boxes)
    offsets = idxs.astype(boxes.dtype) * (max_coordinate + 1.0)
    boxes_for_nms = boxes + offsets[:, None]
    order = jnp.argsort(-scores)
    bs = boxes_for_nms[order]
    padbox = jnp.array([0.0, 0.0, -10.0, -10.0], jnp.float32)
    bp = jnp.concatenate(
        [bs, jnp.broadcast_to(padbox, (NPAD - N, 4))], axis=0)
    x1, y1, x2, y2 = bp[:, 0], bp[:, 1], bp[:, 2], bp[:, 3]
    area = (x2 - x1 + 1.0) * (y2 - y1 + 1.0)

    m, nz = _run_pairs(x1, y1, x2, y2, area)
    rownz = jnp.any(nz[:, :, 0] != 0, axis=0).astype(jnp.int32)
    removed = _run_scan(m, rownz)

    i = jnp.arange(N, dtype=jnp.int32)
    word = (i >> 12) * WPC + (i & (WPC - 1))
    bit = ((i & (CB - 1)) >> 7).astype(jnp.uint32)
    keep_sorted = ((removed[word] >> bit) & jnp.uint32(1)) == 0
    kept_boxes = jnp.where(keep_sorted[:, None], boxes[order], 0.0)
    kept_scores = jnp.where(keep_sorted, scores[order], 0.0)
    return kept_boxes, kept_scores, keep_sorted


# speedup vs baseline: 64.1375x; 1.0248x over previous
import functools

import jax
import jax.numpy as jnp
from jax import lax
from jax.experimental import pallas as pl
from jax.experimental.pallas import tpu as pltpu
from jax.experimental.pallas import tpu_sc as plsc

N = 20000
NPAD = 20480
THR = 0.5

RB = 512
CB = 4096
WPC = CB // 32
NW = NPAD // 32
NRB = NPAD // RB
NCB = NPAD // CB

GB = 64


def _pairs_body(x1r, y1r, x2r, y2r, ar, x1c, y1c, x2c, y2c, ac, m_ref, nz_ref):
    r = pl.program_id(0)
    c = pl.program_id(1)
    full_upper = c * CB > r * RB + (RB - 1)
    full_lower = c * CB + (CB - 1) <= r * RB

    def emit(masked):
        xx1 = jnp.maximum(x1r[...], x1c[...])
        yy1 = jnp.maximum(y1r[...], y1c[...])
        xx2 = jnp.minimum(x2r[...], x2c[...])
        yy2 = jnp.minimum(y2r[...], y2c[...])
        w = jnp.maximum(xx2 - xx1 + 1.0, 0.0)
        h = jnp.maximum(yy2 - yy1 + 1.0, 0.0)
        inter = w * h
        o = inter / (ar[...] + ac[...] - inter)
        sup = o > THR
        if masked:
            ivec = lax.broadcasted_iota(jnp.int32, (RB, CB), 0) + r * RB
            jvec = lax.broadcasted_iota(jnp.int32, (RB, CB), 1) + c * CB
            sup = sup & (jvec > ivec)
        acc = jnp.zeros((RB, WPC), jnp.uint32)
        for b in range(32):
            acc = acc | jnp.where(
                sup[:, b * WPC:(b + 1) * WPC],
                jnp.uint32(1 << b), jnp.uint32(0))
        m_ref[...] = acc
        nz_ref[...] = jnp.any(acc != 0, axis=1, keepdims=True).astype(
            jnp.int32).reshape(1, RB, 1)

    @pl.when(full_upper)
    def _upper():
        emit(masked=False)

    @pl.when(jnp.logical_not(full_upper | full_lower))
    def _diag():
        emit(masked=True)

    @pl.when(full_lower)
    def _zero():
        m_ref[...] = jnp.zeros((RB, WPC), jnp.uint32)
        nz_ref[...] = jnp.zeros((1, RB, 1), jnp.int32)


def _run_pairs(x1, y1, x2, y2, area, interpret=False):
    row = lambda a: a.reshape(NPAD, 1)
    col = lambda a: a.reshape(1, NPAD)
    rspec = pl.BlockSpec((RB, 1), lambda r, c: (r, 0))
    cspec = pl.BlockSpec((1, CB), lambda r, c: (0, c))
    return pl.pallas_call(
        _pairs_body,
        grid=(NRB, NCB),
        in_specs=[rspec] * 5 + [cspec] * 5,
        out_specs=[
            pl.BlockSpec((RB, WPC), lambda r, c: (r, c)),
            pl.BlockSpec((1, RB, 1), lambda r, c: (c, r, 0)),
        ],
        out_shape=[
            jax.ShapeDtypeStruct((NPAD, NW), jnp.uint32),
            jax.ShapeDtypeStruct((NCB, NPAD, 1), jnp.int32),
        ],
        interpret=interpret,
    )(row(x1), row(y1), row(x2), row(y2), row(area),
      col(x1), col(y1), col(x2), col(y2), col(area))


def _scan_body(m_hbm, rnz_hbm, out_hbm, idxbuf, rnzbuf, idxbatch, rowbuf,
               removed, sem):
    cid = lax.axis_index("c")
    sid = lax.axis_index("s")

    @pl.when(jnp.logical_and(cid == 0, sid == 0))
    def _():
        pltpu.sync_copy(rnz_hbm, rnzbuf)

        zero16 = jnp.zeros((16,), jnp.uint32)

        def zb(w, carry):
            removed[pl.ds(w * 16, 16)] = zero16
            return carry

        lax.fori_loop(0, (NW + 16) // 16, zb, 0)

        def cb(v, cnt):
            vec = rnzbuf[pl.ds(v * 16, 16)]
            mask = vec != 0
            ids = lax.iota(jnp.int32, 16) + v * 16
            mi = mask.astype(jnp.int32)
            pos = cnt + plsc.cumsum(mi) - 1
            plsc.store_scatter(idxbuf, [pos], ids, mask=mask)
            return cnt + jnp.sum(mi)

        cnt = lax.fori_loop(0, NPAD // 16, cb, jnp.int32(0))

        idxbuf[pl.ds(cnt, 16)] = jnp.zeros((16,), jnp.int32)
        idxbuf[pl.ds(cnt + 16, 16)] = jnp.zeros((16,), jnp.int32)

        nb = (cnt + GB - 1) // GB

        def stage(t, buf):
            for h in range(GB // 16):
                idxbatch[buf, pl.ds(h * 16, 16)] = (
                    idxbuf[pl.ds(t * GB + h * 16, 16)])
            pltpu.async_copy(
                m_hbm.at[idxbatch.at[buf]], rowbuf.at[buf], sem.at[buf])

        @pl.when(nb > 0)
        def _prime():
            stage(0, 0)

        def batch(t, carry):
            buf = lax.rem(t, 2)

            @pl.when(t + 1 < nb)
            def _ahead():
                stage(t + 1, 1 - buf)

            pltpu.make_async_copy(
                m_hbm.at[idxbatch.at[buf]], rowbuf.at[buf], sem.at[buf]).wait()
            kmax = jnp.minimum(GB, cnt - t * GB)

            def sb(k, c2):
                i = idxbuf[pl.ds(t * GB + k, 16)][0]
                word = (i >> 12) * WPC + (i & (WPC - 1))
                bit = ((i & (CB - 1)) >> 7).astype(jnp.uint32)
                wv = removed[pl.ds(word, 16)][0]
                alive = ((wv >> bit) & jnp.uint32(1)) == 0

                @pl.when(alive)
                def _or():
                    ws = (i >> 12) * (WPC // 16)

                    def ob(w, c3):
                        sl = pl.ds(w * 16, 16)
                        removed[sl] = removed[sl] | rowbuf[buf, k, sl]
                        return c3

                    lax.fori_loop(ws, NW // 16, ob, 0)

                return c2

            lax.fori_loop(0, kmax, sb, 0)
            return carry

        lax.fori_loop(0, nb, batch, 0)
        pltpu.sync_copy(removed.at[pl.ds(0, NW)], out_hbm)


def _run_scan(m, rownz):
    mesh = plsc.VectorSubcoreMesh(core_axis_name="c", subcore_axis_name="s")
    fn = pl.kernel(
        _scan_body,
        out_type=jax.ShapeDtypeStruct((NW,), jnp.uint32),
        mesh=mesh,
        compiler_params=pltpu.CompilerParams(needs_layout_passes=False),
        scratch_types=[
            pltpu.VMEM((NPAD + 32,), jnp.int32),
            pltpu.VMEM((NPAD,), jnp.int32),
            pltpu.VMEM((2, GB), jnp.int32),
            pltpu.VMEM((2, GB, NW), jnp.uint32),
            pltpu.VMEM((NW + 16,), jnp.uint32),
            pltpu.SemaphoreType.DMA((2,)),
        ],
    )
    return fn(m, rownz)


def kernel(boxes, scores, idxs):
    max_coordinate = jnp.max(boxes)
    offsets = idxs.astype(boxes.dtype) * (max_coordinate + 1.0)
    boxes_for_nms = boxes + offsets[:, None]
    order = jnp.argsort(-scores)
    bs = boxes_for_nms[order]
    padbox = jnp.array([0.0, 0.0, -10.0, -10.0], jnp.float32)
    bp = jnp.concatenate(
        [bs, jnp.broadcast_to(padbox, (NPAD - N, 4))], axis=0)
    x1, y1, x2, y2 = bp[:, 0], bp[:, 1], bp[:, 2], bp[:, 3]
    area = (x2 - x1 + 1.0) * (y2 - y1 + 1.0)

    m, nz = _run_pairs(x1, y1, x2, y2, area)
    rownz = jnp.any(nz[:, :, 0] != 0, axis=0).astype(jnp.int32)
    removed = _run_scan(m, rownz)

    i = jnp.arange(N, dtype=jnp.int32)
    word = (i >> 12) * WPC + (i & (WPC - 1))
    bit = ((i & (CB - 1)) >> 7).astype(jnp.uint32)
    keep_sorted = ((removed[word] >> bit) & jnp.uint32(1)) == 0
    kept_boxes = jnp.where(keep_sorted[:, None], boxes[order], 0.0)
    kept_scores = jnp.where(keep_sorted, scores[order], 0.0)
    return kept_boxes, kept_scores, keep_sorted


# speedup vs baseline: 64.2046x; 1.0010x over previous
import functools

import jax
import jax.numpy as jnp
from jax import lax
from jax.experimental import pallas as pl
from jax.experimental.pallas import tpu as pltpu
from jax.experimental.pallas import tpu_sc as plsc

N = 20000
NPAD = 20480
THR = 0.5

RB = 512
CB = 4096
WPC = CB // 32
NW = NPAD // 32
NRB = NPAD // RB
NCB = NPAD // CB

GB = 64


def _pairs_body(x1r, y1r, x2r, y2r, ar, x1c, y1c, x2c, y2c, ac, m_ref, nz_ref):
    r = pl.program_id(0)
    c = pl.program_id(1)
    full_upper = c * CB > r * RB + (RB - 1)
    full_lower = c * CB + (CB - 1) <= r * RB

    def emit(masked):
        xx1 = jnp.maximum(x1r[...], x1c[...])
        yy1 = jnp.maximum(y1r[...], y1c[...])
        xx2 = jnp.minimum(x2r[...], x2c[...])
        yy2 = jnp.minimum(y2r[...], y2c[...])
        w = jnp.maximum(xx2 - xx1 + 1.0, 0.0)
        h = jnp.maximum(yy2 - yy1 + 1.0, 0.0)
        inter = w * h
        o = inter / (ar[...] + ac[...] - inter)
        sup = o > THR
        if masked:
            ivec = lax.broadcasted_iota(jnp.int32, (RB, CB), 0) + r * RB
            jvec = lax.broadcasted_iota(jnp.int32, (RB, CB), 1) + c * CB
            sup = sup & (jvec > ivec)
        acc = jnp.zeros((RB, WPC), jnp.uint32)
        for b in range(32):
            acc = acc | jnp.where(
                sup[:, b * WPC:(b + 1) * WPC],
                jnp.uint32(1 << b), jnp.uint32(0))
        m_ref[...] = acc
        nz_ref[...] = jnp.any(acc != 0, axis=1, keepdims=True).astype(
            jnp.int32).reshape(1, RB, 1)

    @pl.when(full_upper)
    def _upper():
        emit(masked=False)

    @pl.when(jnp.logical_not(full_upper | full_lower))
    def _diag():
        emit(masked=True)

    @pl.when(full_lower)
    def _zero():
        m_ref[...] = jnp.zeros((RB, WPC), jnp.uint32)
        nz_ref[...] = jnp.zeros((1, RB, 1), jnp.int32)


def _run_pairs(x1, y1, x2, y2, area, interpret=False):
    row = lambda a: a.reshape(NPAD, 1)
    col = lambda a: a.reshape(1, NPAD)
    rspec = pl.BlockSpec((RB, 1), lambda r, c: (r, 0))
    cspec = pl.BlockSpec((1, CB), lambda r, c: (0, c))
    return pl.pallas_call(
        _pairs_body,
        grid=(NRB, NCB),
        in_specs=[rspec] * 5 + [cspec] * 5,
        out_specs=[
            pl.BlockSpec((RB, WPC), lambda r, c: (r, c)),
            pl.BlockSpec((1, RB, 1), lambda r, c: (c, r, 0)),
        ],
        out_shape=[
            jax.ShapeDtypeStruct((NPAD, NW), jnp.uint32),
            jax.ShapeDtypeStruct((NCB, NPAD, 1), jnp.int32),
        ],
        interpret=interpret,
    )(row(x1), row(y1), row(x2), row(y2), row(area),
      col(x1), col(y1), col(x2), col(y2), col(area))


def _scan_body(m_hbm, rnz_hbm, out_hbm, idxbuf, rnzbuf, idxbatch, rowbuf,
               removed, sem):
    cid = lax.axis_index("c")
    sid = lax.axis_index("s")

    @pl.when(jnp.logical_and(cid == 0, sid == 0))
    def _():
        pltpu.sync_copy(rnz_hbm, rnzbuf)

        zero16 = jnp.zeros((16,), jnp.uint32)

        def zb(w, carry):
            removed[pl.ds(w * 16, 16)] = zero16
            return carry

        lax.fori_loop(0, (NW + 16) // 16, zb, 0)

        def cb(v, cnt):
            vec = rnzbuf[pl.ds(v * 16, 16)]
            mask = vec != 0
            ids = lax.iota(jnp.int32, 16) + v * 16
            mi = mask.astype(jnp.int32)
            pos = cnt + plsc.cumsum(mi) - 1
            plsc.store_scatter(idxbuf, [pos], ids, mask=mask)
            return cnt + jnp.sum(mi)

        cnt = lax.fori_loop(0, NPAD // 16, cb, jnp.int32(0))

        idxbuf[pl.ds(cnt, 16)] = jnp.zeros((16,), jnp.int32)
        idxbuf[pl.ds(cnt + 16, 16)] = jnp.zeros((16,), jnp.int32)

        nb = (cnt + GB - 1) // GB

        def stage(t, buf):
            for h in range(GB // 16):
                idxbatch[buf, pl.ds(h * 16, 16)] = (
                    idxbuf[pl.ds(t * GB + h * 16, 16)])
            pltpu.async_copy(
                m_hbm.at[idxbatch.at[buf]], rowbuf.at[buf], sem.at[buf])

        @pl.when(nb > 0)
        def _prime():
            stage(0, 0)

        def batch(t, carry):
            buf = lax.rem(t, 2)

            @pl.when(t + 1 < nb)
            def _ahead():
                stage(t + 1, 1 - buf)

            pltpu.make_async_copy(
                m_hbm.at[idxbatch.at[buf]], rowbuf.at[buf], sem.at[buf]).wait()
            kmax = jnp.minimum(GB, cnt - t * GB)

            def sb(k, c2):
                i = idxbuf[pl.ds(t * GB + k, 16)][0]
                word = (i >> 12) * WPC + (i & (WPC - 1))
                bit = ((i & (CB - 1)) >> 7).astype(jnp.uint32)
                wv = removed[pl.ds(word, 16)][0]
                alive = ((wv >> bit) & jnp.uint32(1)) == 0

                @pl.when(alive)
                def _or():
                    ws4 = (i >> 12) * (WPC // 64)

                    def ob(w4, c3):
                        for u in range(4):
                            sl = pl.ds(w4 * 64 + u * 16, 16)
                            removed[sl] = removed[sl] | rowbuf[buf, k, sl]
                        return c3

                    lax.fori_loop(ws4, NW // 64, ob, 0)

                return c2

            lax.fori_loop(0, kmax, sb, 0)
            return carry

        lax.fori_loop(0, nb, batch, 0)
        pltpu.sync_copy(removed.at[pl.ds(0, NW)], out_hbm)


def _run_scan(m, rownz):
    mesh = plsc.VectorSubcoreMesh(core_axis_name="c", subcore_axis_name="s")
    fn = pl.kernel(
        _scan_body,
        out_type=jax.ShapeDtypeStruct((NW,), jnp.uint32),
        mesh=mesh,
        compiler_params=pltpu.CompilerParams(needs_layout_passes=False),
        scratch_types=[
            pltpu.VMEM((NPAD + 32,), jnp.int32),
            pltpu.VMEM((NPAD,), jnp.int32),
            pltpu.VMEM((2, GB), jnp.int32),
            pltpu.VMEM((2, GB, NW), jnp.uint32),
            pltpu.VMEM((NW + 16,), jnp.uint32),
            pltpu.SemaphoreType.DMA((2,)),
        ],
    )
    return fn(m, rownz)


def kernel(boxes, scores, idxs):
    max_coordinate = jnp.max(boxes)
    offsets = idxs.astype(boxes.dtype) * (max_coordinate + 1.0)
    boxes_for_nms = boxes + offsets[:, None]
    order = jnp.argsort(-scores)
    bs = boxes_for_nms[order]
    padbox = jnp.array([0.0, 0.0, -10.0, -10.0], jnp.float32)
    bp = jnp.concatenate(
        [bs, jnp.broadcast_to(padbox, (NPAD - N, 4))], axis=0)
    x1, y1, x2, y2 = bp[:, 0], bp[:, 1], bp[:, 2], bp[:, 3]
    area = (x2 - x1 + 1.0) * (y2 - y1 + 1.0)

    m, nz = _run_pairs(x1, y1, x2, y2, area)
    rownz = jnp.any(nz[:, :, 0] != 0, axis=0).astype(jnp.int32)
    removed = _run_scan(m, rownz)

    i = jnp.arange(N, dtype=jnp.int32)
    word = (i >> 12) * WPC + (i & (WPC - 1))
    bit = ((i & (CB - 1)) >> 7).astype(jnp.uint32)
    keep_sorted = ((removed[word] >> bit) & jnp.uint32(1)) == 0
    kept_boxes = jnp.where(keep_sorted[:, None], boxes[order], 0.0)
    kept_scores = jnp.where(keep_sorted, scores[order], 0.0)
    return kept_boxes, kept_scores, keep_sorted
